# baseline (speedup 1.0000x reference)
import functools

import jax
import jax.numpy as jnp
from jax.experimental import pallas as pl
from jax.experimental.pallas import tpu as pltpu

N = 10000
DR = 128
D = 64
H = 2
MP = 2
SEM = 128
NCLS = 16
E = 320000
T = 5000

_NBLK = 10
_BN = N // _NBLK
_CH = 2500
_NCH = E // _CH


def _proj_kernel(x_ref, w_ref, b_ref, natt_ref, map_ref, f_ref):
    x = x_ref[...]
    w = w_ref[...]
    proj = jnp.dot(x, w, preferred_element_type=jnp.float32) + b_ref[0, :][None, :]
    mask = map_ref[...] == 0
    feat = jnp.where(mask, proj, 0.0)
    natt = natt_ref[...]
    a_dst = natt[:, 0:D]
    a_src = natt[:, D:2 * D]
    sd = jax.lax.dot_general(feat, a_dst, (((1,), (1,)), ((), ())),
                             preferred_element_type=jnp.float32)
    ss = jax.lax.dot_general(feat, a_src, (((1,), (1,)), ((), ())),
                             preferred_element_type=jnp.float32)
    f_ref[...] = jnp.concatenate([feat, sd, ss], axis=1)


def _scatter_kernel(f_ref, cur_ref, nbr_ref, acc_a_ref, acc_b_ref):
    m = pl.program_id(0)

    @pl.when(pl.program_id(1) == 0)
    def _init():
        acc_a_ref[...] = jnp.zeros_like(acc_a_ref)
        acc_b_ref[...] = jnp.zeros_like(acc_b_ref)

    is_m0 = m == 0
    z = jnp.float32(0.0)

    def body(i, _):
        c = cur_ref[0, 0, i]
        nb = nbr_ref[0, 0, i]
        rowc = f_ref[c, :]
        rown = f_ref[nb, :]
        a0 = jnp.where(is_m0, rowc[D] + rown[D + 4], rowc[D + 2] + rown[D + 6])
        a1 = jnp.where(is_m0, rowc[D + 1] + rown[D + 5], rowc[D + 3] + rown[D + 7])
        a0 = jnp.where(a0 >= 0, a0, 0.01 * a0)
        a1 = jnp.where(a1 >= 0, a1, 0.01 * a1)
        ex0 = jnp.exp(a0)
        ex1 = jnp.exp(a1)
        f = rown[0:D]
        va = jnp.concatenate([ex0 * f, ex1 * f])
        acc_a_ref[0, c, :] = acc_a_ref[0, c, :] + va
        vb = jnp.stack([ex0, ex1, z, z, z, z, z, z])
        acc_b_ref[0, c, :] = acc_b_ref[0, c, :] + vb
        return 0

    jax.lax.fori_loop(0, _CH, body, 0)


def _gather_kernel(acc_a_ref, acc_b_ref, tn_ref, h_ref):
    def body(t, _):
        n = tn_ref[0, 0, t]
        num = acc_a_ref[0, n, :]
        dn = acc_b_ref[0, n, :]
        h0 = num[0:D] / (dn[0] + 1e-16)
        h1 = num[D:2 * D] / (dn[1] + 1e-16)
        h_ref[0, t, :] = jnp.concatenate([h0, h1])
        return 0

    jax.lax.fori_loop(0, T, body, 0)


def _head_kernel(h_ref, wsem_ref, bsem_ref, satt_ref, wcls_ref, bcls_ref,
                 logits_ref, emb_ref):
    h0 = h_ref[0]
    h1 = h_ref[1]
    wsem = wsem_ref[...]
    bsem = bsem_ref[0, :][None, :]
    satt = satt_ref[0, :][None, :]
    a0 = jnp.tanh(jnp.dot(h0, wsem, preferred_element_type=jnp.float32) + bsem)
    a1 = jnp.tanh(jnp.dot(h1, wsem, preferred_element_type=jnp.float32) + bsem)
    l0 = jnp.mean(jnp.sum(satt * a0, axis=1))
    l1 = jnp.mean(jnp.sum(satt * a1, axis=1))
    mx = jnp.maximum(l0, l1)
    e0 = jnp.exp(l0 - mx)
    e1 = jnp.exp(l1 - mx)
    b0 = e0 / (e0 + e1)
    b1 = e1 / (e0 + e1)
    emb = b0 * h0 + b1 * h1
    emb_ref[...] = emb
    logits_ref[...] = (jnp.dot(emb, wcls_ref[...], preferred_element_type=jnp.float32)
                       + bcls_ref[0, :][None, :])


@jax.jit
def kernel(target_nodes, metapath_list, node_type_mapping, node_feature_list,
           W_proj, b_proj, node_attention, W_sem, b_sem, sem_att, W_cls, b_cls):
    x = node_feature_list[0]
    natt4 = node_attention.reshape(MP * H, 2 * D)
    map2 = node_type_mapping.reshape(N, 1)
    b_proj2 = b_proj.reshape(1, D)

    f_table = pl.pallas_call(
        _proj_kernel,
        grid=(_NBLK,),
        in_specs=[
            pl.BlockSpec((_BN, DR), lambda i: (i, 0)),
            pl.BlockSpec((DR, D), lambda i: (0, 0)),
            pl.BlockSpec((1, D), lambda i: (0, 0)),
            pl.BlockSpec((MP * H, 2 * D), lambda i: (0, 0)),
            pl.BlockSpec((_BN, 1), lambda i: (i, 0)),
        ],
        out_specs=pl.BlockSpec((_BN, D + 2 * MP * H), lambda i: (i, 0)),
        out_shape=jax.ShapeDtypeStruct((N, D + 2 * MP * H), jnp.float32),
    )(x, W_proj, b_proj2, natt4, map2)

    cur = metapath_list[:, :, 1].reshape(MP * _NCH, 1, _CH)
    nbr = metapath_list[:, :, 0].reshape(MP * _NCH, 1, _CH)

    acc_a, acc_b = pl.pallas_call(
        _scatter_kernel,
        grid=(MP, _NCH),
        in_specs=[
            pl.BlockSpec((N, D + 2 * MP * H), lambda m, e: (0, 0)),
            pl.BlockSpec((1, 1, _CH), lambda m, e: (m * _NCH + e, 0, 0),
                         memory_space=pltpu.SMEM),
            pl.BlockSpec((1, 1, _CH), lambda m, e: (m * _NCH + e, 0, 0),
                         memory_space=pltpu.SMEM),
        ],
        out_specs=[
            pl.BlockSpec((1, N, 2 * D), lambda m, e: (m, 0, 0)),
            pl.BlockSpec((1, N, 8), lambda m, e: (m, 0, 0)),
        ],
        out_shape=[
            jax.ShapeDtypeStruct((MP, N, 2 * D), jnp.float32),
            jax.ShapeDtypeStruct((MP, N, 8), jnp.float32),
        ],
    )(f_table, cur, nbr)

    tn3 = target_nodes.reshape(1, 1, T)
    h_mp = pl.pallas_call(
        _gather_kernel,
        grid=(MP,),
        in_specs=[
            pl.BlockSpec((1, N, 2 * D), lambda m: (m, 0, 0)),
            pl.BlockSpec((1, N, 8), lambda m: (m, 0, 0)),
            pl.BlockSpec((1, 1, T), lambda m: (0, 0, 0),
                         memory_space=pltpu.SMEM),
        ],
        out_specs=pl.BlockSpec((1, T, 2 * D), lambda m: (m, 0, 0)),
        out_shape=jax.ShapeDtypeStruct((MP, T, 2 * D), jnp.float32),
    )(acc_a, acc_b, tn3)

    logits, emb = pl.pallas_call(
        _head_kernel,
        grid=(1,),
        in_specs=[
            pl.BlockSpec((MP, T, 2 * D), lambda i: (0, 0, 0)),
            pl.BlockSpec((H * D, SEM), lambda i: (0, 0)),
            pl.BlockSpec((1, SEM), lambda i: (0, 0)),
            pl.BlockSpec((1, SEM), lambda i: (0, 0)),
            pl.BlockSpec((H * D, NCLS), lambda i: (0, 0)),
            pl.BlockSpec((1, NCLS), lambda i: (0, 0)),
        ],
        out_specs=[
            pl.BlockSpec((T, NCLS), lambda i: (0, 0)),
            pl.BlockSpec((T, 2 * D), lambda i: (0, 0)),
        ],
        out_shape=[
            jax.ShapeDtypeStruct((T, NCLS), jnp.float32),
            jax.ShapeDtypeStruct((T, 2 * D), jnp.float32),
        ],
    )(h_mp, W_sem, b_sem.reshape(1, SEM), sem_att, W_cls, b_cls.reshape(1, NCLS))

    return (logits, emb)


# speedup vs baseline: 2.4899x; 2.4899x over previous
import functools

import jax
import jax.numpy as jnp
from jax.experimental import pallas as pl
from jax.experimental.pallas import tpu as pltpu

N = 10000
DR = 128
D = 64
H = 2
MP = 2
SEM = 128
NCLS = 16
E = 320000
T = 5000

_NBLK = 10
_BN = N // _NBLK
_CH = 2500
_NCH = E // _CH
_UNROLL = 4
_FW = D + 2 * H


def _proj_kernel(x_ref, w_ref, b_ref, natt_ref, map_ref, f_ref):
    m = pl.program_id(0)
    x = x_ref[...]
    w = w_ref[...]
    proj = jnp.dot(x, w, preferred_element_type=jnp.float32) + b_ref[0, :][None, :]
    mask = map_ref[...] == 0
    feat = jnp.where(mask, proj, 0.0)
    natt = natt_ref[pl.ds(2 * m, H), :]
    a_dst = natt[:, 0:D]
    a_src = natt[:, D:2 * D]
    sd = jax.lax.dot_general(feat, a_dst, (((1,), (1,)), ((), ())),
                             preferred_element_type=jnp.float32)
    ss = jax.lax.dot_general(feat, a_src, (((1,), (1,)), ((), ())),
                             preferred_element_type=jnp.float32)
    f_ref[0] = jnp.concatenate([feat, sd, ss], axis=1)


def _scatter_kernel(f_ref, cur_ref, nbr_ref, acc_a_ref, acc_b_ref):
    m = pl.program_id(0)

    @pl.when(pl.program_id(1) == 0)
    def _init():
        acc_a_ref[...] = jnp.zeros_like(acc_a_ref)
        acc_b_ref[...] = jnp.zeros_like(acc_b_ref)

    z = jnp.float32(0.0)

    def edge(i):
        c = cur_ref[0, 0, i]
        nb = nbr_ref[0, 0, i]
        rowc = f_ref[0, c, :]
        rown = f_ref[0, nb, :]
        a0 = rowc[D] + rown[D + 2]
        a1 = rowc[D + 1] + rown[D + 3]
        a0 = jnp.where(a0 >= 0, a0, 0.01 * a0)
        a1 = jnp.where(a1 >= 0, a1, 0.01 * a1)
        ex0 = jnp.exp(a0)
        ex1 = jnp.exp(a1)
        f = rown[0:D]
        va = jnp.concatenate([ex0 * f, ex1 * f])
        acc_a_ref[0, c, :] = acc_a_ref[0, c, :] + va
        vb = jnp.stack([ex0, ex1, z, z, z, z, z, z])
        acc_b_ref[0, c, :] = acc_b_ref[0, c, :] + vb

    def body(j, _):
        for k in range(_UNROLL):
            edge(_UNROLL * j + k)
        return 0

    jax.lax.fori_loop(0, _CH // _UNROLL, body, 0)


def _gather_kernel(acc_a_ref, acc_b_ref, tn_ref, h_ref):
    def body(t, _):
        n = tn_ref[0, 0, t]
        num = acc_a_ref[0, n, :]
        dn = acc_b_ref[0, n, :]
        h0 = num[0:D] / (dn[0] + 1e-16)
        h1 = num[D:2 * D] / (dn[1] + 1e-16)
        h_ref[0, t, :] = jnp.concatenate([h0, h1])
        return 0

    jax.lax.fori_loop(0, T, body, 0)


def _head_kernel(h_ref, wsem_ref, bsem_ref, satt_ref, wcls_ref, bcls_ref,
                 logits_ref, emb_ref):
    h0 = h_ref[0]
    h1 = h_ref[1]
    wsem = wsem_ref[...]
    bsem = bsem_ref[0, :][None, :]
    satt = satt_ref[0, :][None, :]
    a0 = jnp.tanh(jnp.dot(h0, wsem, preferred_element_type=jnp.float32) + bsem)
    a1 = jnp.tanh(jnp.dot(h1, wsem, preferred_element_type=jnp.float32) + bsem)
    l0 = jnp.mean(jnp.sum(satt * a0, axis=1))
    l1 = jnp.mean(jnp.sum(satt * a1, axis=1))
    mx = jnp.maximum(l0, l1)
    e0 = jnp.exp(l0 - mx)
    e1 = jnp.exp(l1 - mx)
    b0 = e0 / (e0 + e1)
    b1 = e1 / (e0 + e1)
    emb = b0 * h0 + b1 * h1
    emb_ref[...] = emb
    logits_ref[...] = (jnp.dot(emb, wcls_ref[...], preferred_element_type=jnp.float32)
                       + bcls_ref[0, :][None, :])


@jax.jit
def kernel(target_nodes, metapath_list, node_type_mapping, node_feature_list,
           W_proj, b_proj, node_attention, W_sem, b_sem, sem_att, W_cls, b_cls):
    x = node_feature_list[0]
    natt4 = node_attention.reshape(MP * H, 2 * D)
    map2 = node_type_mapping.reshape(N, 1)
    b_proj2 = b_proj.reshape(1, D)

    f_table = pl.pallas_call(
        _proj_kernel,
        grid=(MP, _NBLK),
        in_specs=[
            pl.BlockSpec((_BN, DR), lambda m, i: (i, 0)),
            pl.BlockSpec((DR, D), lambda m, i: (0, 0)),
            pl.BlockSpec((1, D), lambda m, i: (0, 0)),
            pl.BlockSpec((MP * H, 2 * D), lambda m, i: (0, 0)),
            pl.BlockSpec((_BN, 1), lambda m, i: (i, 0)),
        ],
        out_specs=pl.BlockSpec((1, _BN, _FW), lambda m, i: (m, i, 0)),
        out_shape=jax.ShapeDtypeStruct((MP, N, _FW), jnp.float32),
        compiler_params=pltpu.CompilerParams(
            dimension_semantics=("parallel", "parallel")),
    )(x, W_proj, b_proj2, natt4, map2)

    cur = metapath_list[:, :, 1].reshape(MP * _NCH, 1, _CH)
    nbr = metapath_list[:, :, 0].reshape(MP * _NCH, 1, _CH)

    acc_a, acc_b = pl.pallas_call(
        _scatter_kernel,
        grid=(MP, _NCH),
        in_specs=[
            pl.BlockSpec((1, N, _FW), lambda m, e: (m, 0, 0)),
            pl.BlockSpec((1, 1, _CH), lambda m, e: (m * _NCH + e, 0, 0),
                         memory_space=pltpu.SMEM),
            pl.BlockSpec((1, 1, _CH), lambda m, e: (m * _NCH + e, 0, 0),
                         memory_space=pltpu.SMEM),
        ],
        out_specs=[
            pl.BlockSpec((1, N, 2 * D), lambda m, e: (m, 0, 0)),
            pl.BlockSpec((1, N, 8), lambda m, e: (m, 0, 0)),
        ],
        out_shape=[
            jax.ShapeDtypeStruct((MP, N, 2 * D), jnp.float32),
            jax.ShapeDtypeStruct((MP, N, 8), jnp.float32),
        ],
        compiler_params=pltpu.CompilerParams(
            dimension_semantics=("parallel", "arbitrary")),
    )(f_table, cur, nbr)

    tn3 = target_nodes.reshape(1, 1, T)
    h_mp = pl.pallas_call(
        _gather_kernel,
        grid=(MP,),
        in_specs=[
            pl.BlockSpec((1, N, 2 * D), lambda m: (m, 0, 0)),
            pl.BlockSpec((1, N, 8), lambda m: (m, 0, 0)),
            pl.BlockSpec((1, 1, T), lambda m: (0, 0, 0),
                         memory_space=pltpu.SMEM),
        ],
        out_specs=pl.BlockSpec((1, T, 2 * D), lambda m: (m, 0, 0)),
        out_shape=jax.ShapeDtypeStruct((MP, T, 2 * D), jnp.float32),
        compiler_params=pltpu.CompilerParams(
            dimension_semantics=("parallel",)),
    )(acc_a, acc_b, tn3)

    logits, emb = pl.pallas_call(
        _head_kernel,
        grid=(1,),
        in_specs=[
            pl.BlockSpec((MP, T, 2 * D), lambda i: (0, 0, 0)),
            pl.BlockSpec((H * D, SEM), lambda i: (0, 0)),
            pl.BlockSpec((1, SEM), lambda i: (0, 0)),
            pl.BlockSpec((1, SEM), lambda i: (0, 0)),
            pl.BlockSpec((H * D, NCLS), lambda i: (0, 0)),
            pl.BlockSpec((1, NCLS), lambda i: (0, 0)),
        ],
        out_specs=[
            pl.BlockSpec((T, NCLS), lambda i: (0, 0)),
            pl.BlockSpec((T, 2 * D), lambda i: (0, 0)),
        ],
        out_shape=[
            jax.ShapeDtypeStruct((T, NCLS), jnp.float32),
            jax.ShapeDtypeStruct((T, 2 * D), jnp.float32),
        ],
    )(h_mp, W_sem, b_sem.reshape(1, SEM), sem_att, W_cls, b_cls.reshape(1, NCLS))

    return (logits, emb)


# speedup vs baseline: 3.4990x; 1.4053x over previous
import functools

import jax
import jax.numpy as jnp
from jax.experimental import pallas as pl
from jax.experimental.pallas import tpu as pltpu

N = 10000
DR = 128
D = 64
H = 2
MP = 2
SEM = 128
NCLS = 16
E = 320000
T = 5000

_NBLK = 10
_BN = N // _NBLK
_CH = 2500
_NCH = E // _CH
_UNROLL = 10
_FW = D + 2 * H


def _proj_kernel(x_ref, w_ref, b_ref, natt_ref, map_ref, f_ref):
    m = pl.program_id(0)
    x = x_ref[...]
    w = w_ref[...]
    proj = jnp.dot(x, w, preferred_element_type=jnp.float32) + b_ref[0, :][None, :]
    mask = map_ref[...] == 0
    feat = jnp.where(mask, proj, 0.0)
    natt = natt_ref[pl.ds(2 * m, H), :]
    a_dst = natt[:, 0:D]
    a_src = natt[:, D:2 * D]
    sd = jax.lax.dot_general(feat, a_dst, (((1,), (1,)), ((), ())),
                             preferred_element_type=jnp.float32)
    ss = jax.lax.dot_general(feat, a_src, (((1,), (1,)), ((), ())),
                             preferred_element_type=jnp.float32)
    f_ref[0] = jnp.concatenate([feat, sd, ss], axis=1)


def _scatter_kernel(f_ref, cur_ref, nbr_ref, acc_a_ref, acc_b_ref):
    m = pl.program_id(0)

    @pl.when(pl.program_id(1) == 0)
    def _init():
        acc_a_ref[...] = jnp.zeros_like(acc_a_ref)
        acc_b_ref[...] = jnp.zeros_like(acc_b_ref)

    z = jnp.float32(0.0)

    def edge(i):
        c = cur_ref[0, 0, i]
        nb = nbr_ref[0, 0, i]
        rowc = f_ref[0, c, :]
        rown = f_ref[0, nb, :]
        a0 = rowc[D] + rown[D + 2]
        a1 = rowc[D + 1] + rown[D + 3]
        a0 = jnp.where(a0 >= 0, a0, 0.01 * a0)
        a1 = jnp.where(a1 >= 0, a1, 0.01 * a1)
        ex0 = jnp.exp(a0)
        ex1 = jnp.exp(a1)
        f = rown[0:D]
        va = jnp.concatenate([ex0 * f, ex1 * f])
        acc_a_ref[0, c, :] = acc_a_ref[0, c, :] + va
        vb = jnp.stack([ex0, ex1, z, z, z, z, z, z])
        acc_b_ref[0, c, :] = acc_b_ref[0, c, :] + vb

    def body(j, _):
        for k in range(_UNROLL):
            edge(_UNROLL * j + k)
        return 0

    jax.lax.fori_loop(0, _CH // _UNROLL, body, 0)


def _gather_kernel(acc_a_ref, acc_b_ref, tn_ref, h_ref):
    def body(t, _):
        n = tn_ref[0, 0, t]
        num = acc_a_ref[0, n, :]
        dn = acc_b_ref[0, n, :]
        h0 = num[0:D] / (dn[0] + 1e-16)
        h1 = num[D:2 * D] / (dn[1] + 1e-16)
        h_ref[0, t, :] = jnp.concatenate([h0, h1])
        return 0

    jax.lax.fori_loop(0, T, body, 0)


def _head_kernel(h_ref, wsem_ref, bsem_ref, satt_ref, wcls_ref, bcls_ref,
                 logits_ref, emb_ref):
    h0 = h_ref[0]
    h1 = h_ref[1]
    wsem = wsem_ref[...]
    bsem = bsem_ref[0, :][None, :]
    satt = satt_ref[0, :][None, :]
    a0 = jnp.tanh(jnp.dot(h0, wsem, preferred_element_type=jnp.float32) + bsem)
    a1 = jnp.tanh(jnp.dot(h1, wsem, preferred_element_type=jnp.float32) + bsem)
    l0 = jnp.mean(jnp.sum(satt * a0, axis=1))
    l1 = jnp.mean(jnp.sum(satt * a1, axis=1))
    mx = jnp.maximum(l0, l1)
    e0 = jnp.exp(l0 - mx)
    e1 = jnp.exp(l1 - mx)
    b0 = e0 / (e0 + e1)
    b1 = e1 / (e0 + e1)
    emb = b0 * h0 + b1 * h1
    emb_ref[...] = emb
    logits_ref[...] = (jnp.dot(emb, wcls_ref[...], preferred_element_type=jnp.float32)
                       + bcls_ref[0, :][None, :])


@jax.jit
def kernel(target_nodes, metapath_list, node_type_mapping, node_feature_list,
           W_proj, b_proj, node_attention, W_sem, b_sem, sem_att, W_cls, b_cls):
    x = node_feature_list[0]
    natt4 = node_attention.reshape(MP * H, 2 * D)
    map2 = node_type_mapping.reshape(N, 1)
    b_proj2 = b_proj.reshape(1, D)

    f_table = pl.pallas_call(
        _proj_kernel,
        grid=(MP, _NBLK),
        in_specs=[
            pl.BlockSpec((_BN, DR), lambda m, i: (i, 0)),
            pl.BlockSpec((DR, D), lambda m, i: (0, 0)),
            pl.BlockSpec((1, D), lambda m, i: (0, 0)),
            pl.BlockSpec((MP * H, 2 * D), lambda m, i: (0, 0)),
            pl.BlockSpec((_BN, 1), lambda m, i: (i, 0)),
        ],
        out_specs=pl.BlockSpec((1, _BN, _FW), lambda m, i: (m, i, 0)),
        out_shape=jax.ShapeDtypeStruct((MP, N, _FW), jnp.float32),
        compiler_params=pltpu.CompilerParams(
            dimension_semantics=("parallel", "parallel")),
    )(x, W_proj, b_proj2, natt4, map2)

    cur = metapath_list[:, :, 1].reshape(MP * _NCH, 1, _CH)
    nbr = metapath_list[:, :, 0].reshape(MP * _NCH, 1, _CH)

    acc_a, acc_b = pl.pallas_call(
        _scatter_kernel,
        grid=(MP, _NCH),
        in_specs=[
            pl.BlockSpec((1, N, _FW), lambda m, e: (m, 0, 0)),
            pl.BlockSpec((1, 1, _CH), lambda m, e: (m * _NCH + e, 0, 0),
                         memory_space=pltpu.SMEM),
            pl.BlockSpec((1, 1, _CH), lambda m, e: (m * _NCH + e, 0, 0),
                         memory_space=pltpu.SMEM),
        ],
        out_specs=[
            pl.BlockSpec((1, N, 2 * D), lambda m, e: (m, 0, 0)),
            pl.BlockSpec((1, N, 8), lambda m, e: (m, 0, 0)),
        ],
        out_shape=[
            jax.ShapeDtypeStruct((MP, N, 2 * D), jnp.float32),
            jax.ShapeDtypeStruct((MP, N, 8), jnp.float32),
        ],
        compiler_params=pltpu.CompilerParams(
            dimension_semantics=("parallel", "arbitrary")),
    )(f_table, cur, nbr)

    tn3 = target_nodes.reshape(1, 1, T)
    h_mp = pl.pallas_call(
        _gather_kernel,
        grid=(MP,),
        in_specs=[
            pl.BlockSpec((1, N, 2 * D), lambda m: (m, 0, 0)),
            pl.BlockSpec((1, N, 8), lambda m: (m, 0, 0)),
            pl.BlockSpec((1, 1, T), lambda m: (0, 0, 0),
                         memory_space=pltpu.SMEM),
        ],
        out_specs=pl.BlockSpec((1, T, 2 * D), lambda m: (m, 0, 0)),
        out_shape=jax.ShapeDtypeStruct((MP, T, 2 * D), jnp.float32),
        compiler_params=pltpu.CompilerParams(
            dimension_semantics=("parallel",)),
    )(acc_a, acc_b, tn3)

    logits, emb = pl.pallas_call(
        _head_kernel,
        grid=(1,),
        in_specs=[
            pl.BlockSpec((MP, T, 2 * D), lambda i: (0, 0, 0)),
            pl.BlockSpec((H * D, SEM), lambda i: (0, 0)),
            pl.BlockSpec((1, SEM), lambda i: (0, 0)),
            pl.BlockSpec((1, SEM), lambda i: (0, 0)),
            pl.BlockSpec((H * D, NCLS), lambda i: (0, 0)),
            pl.BlockSpec((1, NCLS), lambda i: (0, 0)),
        ],
        out_specs=[
            pl.BlockSpec((T, NCLS), lambda i: (0, 0)),
            pl.BlockSpec((T, 2 * D), lambda i: (0, 0)),
        ],
        out_shape=[
            jax.ShapeDtypeStruct((T, NCLS), jnp.float32),
            jax.ShapeDtypeStruct((T, 2 * D), jnp.float32),
        ],
    )(h_mp, W_sem, b_sem.reshape(1, SEM), sem_att, W_cls, b_cls.reshape(1, NCLS))

    return (logits, emb)


# speedup vs baseline: 4.2045x; 1.2016x over previous
import functools

import jax
import jax.numpy as jnp
from jax.experimental import pallas as pl
from jax.experimental.pallas import tpu as pltpu

N = 10000
DR = 128
D = 64
H = 2
MP = 2
SEM = 128
NCLS = 16
E = 320000
T = 5000

_NBLK = 10
_BN = N // _NBLK
_CH = 2500
_NCH = E // _CH
_UNROLL = 25
_FW = D + 2 * H


def _proj_kernel(x_ref, w_ref, b_ref, natt_ref, map_ref, f_ref):
    m = pl.program_id(0)
    x = x_ref[...]
    w = w_ref[...]
    proj = jnp.dot(x, w, preferred_element_type=jnp.float32) + b_ref[0, :][None, :]
    mask = map_ref[...] == 0
    feat = jnp.where(mask, proj, 0.0)
    natt = natt_ref[pl.ds(2 * m, H), :]
    a_dst = natt[:, 0:D]
    a_src = natt[:, D:2 * D]
    sd = jax.lax.dot_general(feat, a_dst, (((1,), (1,)), ((), ())),
                             preferred_element_type=jnp.float32)
    ss = jax.lax.dot_general(feat, a_src, (((1,), (1,)), ((), ())),
                             preferred_element_type=jnp.float32)
    f_ref[0] = jnp.concatenate([feat, sd, ss], axis=1)


def _scatter_kernel(f_ref, cur_ref, nbr_ref, acc_a_ref, acc_b_ref):
    m = pl.program_id(0)

    @pl.when(pl.program_id(1) == 0)
    def _init():
        acc_a_ref[...] = jnp.zeros_like(acc_a_ref)
        acc_b_ref[...] = jnp.zeros_like(acc_b_ref)

    z = jnp.float32(0.0)

    def edge(i):
        c = cur_ref[0, 0, i]
        nb = nbr_ref[0, 0, i]
        rowc = f_ref[0, c, :]
        rown = f_ref[0, nb, :]
        a0 = rowc[D] + rown[D + 2]
        a1 = rowc[D + 1] + rown[D + 3]
        a0 = jnp.where(a0 >= 0, a0, 0.01 * a0)
        a1 = jnp.where(a1 >= 0, a1, 0.01 * a1)
        ex0 = jnp.exp(a0)
        ex1 = jnp.exp(a1)
        f = rown[0:D]
        va = jnp.concatenate([ex0 * f, ex1 * f])
        acc_a_ref[0, c, :] = acc_a_ref[0, c, :] + va
        vb = jnp.stack([ex0, ex1, z, z, z, z, z, z])
        acc_b_ref[0, c, :] = acc_b_ref[0, c, :] + vb

    def body(j, _):
        for k in range(_UNROLL):
            edge(_UNROLL * j + k)
        return 0

    jax.lax.fori_loop(0, _CH // _UNROLL, body, 0)


def _gather_kernel(acc_a_ref, acc_b_ref, tn_ref, h_ref):
    def body(t, _):
        n = tn_ref[0, 0, t]
        num = acc_a_ref[0, n, :]
        dn = acc_b_ref[0, n, :]
        h0 = num[0:D] / (dn[0] + 1e-16)
        h1 = num[D:2 * D] / (dn[1] + 1e-16)
        h_ref[0, t, :] = jnp.concatenate([h0, h1])
        return 0

    jax.lax.fori_loop(0, T, body, 0)


def _head_kernel(h_ref, wsem_ref, bsem_ref, satt_ref, wcls_ref, bcls_ref,
                 logits_ref, emb_ref):
    h0 = h_ref[0]
    h1 = h_ref[1]
    wsem = wsem_ref[...]
    bsem = bsem_ref[0, :][None, :]
    satt = satt_ref[0, :][None, :]
    a0 = jnp.tanh(jnp.dot(h0, wsem, preferred_element_type=jnp.float32) + bsem)
    a1 = jnp.tanh(jnp.dot(h1, wsem, preferred_element_type=jnp.float32) + bsem)
    l0 = jnp.mean(jnp.sum(satt * a0, axis=1))
    l1 = jnp.mean(jnp.sum(satt * a1, axis=1))
    mx = jnp.maximum(l0, l1)
    e0 = jnp.exp(l0 - mx)
    e1 = jnp.exp(l1 - mx)
    b0 = e0 / (e0 + e1)
    b1 = e1 / (e0 + e1)
    emb = b0 * h0 + b1 * h1
    emb_ref[...] = emb
    logits_ref[...] = (jnp.dot(emb, wcls_ref[...], preferred_element_type=jnp.float32)
                       + bcls_ref[0, :][None, :])


@jax.jit
def kernel(target_nodes, metapath_list, node_type_mapping, node_feature_list,
           W_proj, b_proj, node_attention, W_sem, b_sem, sem_att, W_cls, b_cls):
    x = node_feature_list[0]
    natt4 = node_attention.reshape(MP * H, 2 * D)
    map2 = node_type_mapping.reshape(N, 1)
    b_proj2 = b_proj.reshape(1, D)

    f_table = pl.pallas_call(
        _proj_kernel,
        grid=(MP, _NBLK),
        in_specs=[
            pl.BlockSpec((_BN, DR), lambda m, i: (i, 0)),
            pl.BlockSpec((DR, D), lambda m, i: (0, 0)),
            pl.BlockSpec((1, D), lambda m, i: (0, 0)),
            pl.BlockSpec((MP * H, 2 * D), lambda m, i: (0, 0)),
            pl.BlockSpec((_BN, 1), lambda m, i: (i, 0)),
        ],
        out_specs=pl.BlockSpec((1, _BN, _FW), lambda m, i: (m, i, 0)),
        out_shape=jax.ShapeDtypeStruct((MP, N, _FW), jnp.float32),
        compiler_params=pltpu.CompilerParams(
            dimension_semantics=("parallel", "parallel")),
    )(x, W_proj, b_proj2, natt4, map2)

    cur = metapath_list[:, :, 1].reshape(MP * _NCH, 1, _CH)
    nbr = metapath_list[:, :, 0].reshape(MP * _NCH, 1, _CH)

    acc_a, acc_b = pl.pallas_call(
        _scatter_kernel,
        grid=(MP, _NCH),
        in_specs=[
            pl.BlockSpec((1, N, _FW), lambda m, e: (m, 0, 0)),
            pl.BlockSpec((1, 1, _CH), lambda m, e: (m * _NCH + e, 0, 0),
                         memory_space=pltpu.SMEM),
            pl.BlockSpec((1, 1, _CH), lambda m, e: (m * _NCH + e, 0, 0),
                         memory_space=pltpu.SMEM),
        ],
        out_specs=[
            pl.BlockSpec((1, N, 2 * D), lambda m, e: (m, 0, 0)),
            pl.BlockSpec((1, N, 8), lambda m, e: (m, 0, 0)),
        ],
        out_shape=[
            jax.ShapeDtypeStruct((MP, N, 2 * D), jnp.float32),
            jax.ShapeDtypeStruct((MP, N, 8), jnp.float32),
        ],
        compiler_params=pltpu.CompilerParams(
            dimension_semantics=("parallel", "arbitrary")),
    )(f_table, cur, nbr)

    tn3 = target_nodes.reshape(1, 1, T)
    h_mp = pl.pallas_call(
        _gather_kernel,
        grid=(MP,),
        in_specs=[
            pl.BlockSpec((1, N, 2 * D), lambda m: (m, 0, 0)),
            pl.BlockSpec((1, N, 8), lambda m: (m, 0, 0)),
            pl.BlockSpec((1, 1, T), lambda m: (0, 0, 0),
                         memory_space=pltpu.SMEM),
        ],
        out_specs=pl.BlockSpec((1, T, 2 * D), lambda m: (m, 0, 0)),
        out_shape=jax.ShapeDtypeStruct((MP, T, 2 * D), jnp.float32),
        compiler_params=pltpu.CompilerParams(
            dimension_semantics=("parallel",)),
    )(acc_a, acc_b, tn3)

    logits, emb = pl.pallas_call(
        _head_kernel,
        grid=(1,),
        in_specs=[
            pl.BlockSpec((MP, T, 2 * D), lambda i: (0, 0, 0)),
            pl.BlockSpec((H * D, SEM), lambda i: (0, 0)),
            pl.BlockSpec((1, SEM), lambda i: (0, 0)),
            pl.BlockSpec((1, SEM), lambda i: (0, 0)),
            pl.BlockSpec((H * D, NCLS), lambda i: (0, 0)),
            pl.BlockSpec((1, NCLS), lambda i: (0, 0)),
        ],
        out_specs=[
            pl.BlockSpec((T, NCLS), lambda i: (0, 0)),
            pl.BlockSpec((T, 2 * D), lambda i: (0, 0)),
        ],
        out_shape=[
            jax.ShapeDtypeStruct((T, NCLS), jnp.float32),
            jax.ShapeDtypeStruct((T, 2 * D), jnp.float32),
        ],
    )(h_mp, W_sem, b_sem.reshape(1, SEM), sem_att, W_cls, b_cls.reshape(1, NCLS))

    return (logits, emb)


# speedup vs baseline: 4.4808x; 1.0657x over previous
import functools

import jax
import jax.numpy as jnp
from jax.experimental import pallas as pl
from jax.experimental.pallas import tpu as pltpu

N = 10000
DR = 128
D = 64
H = 2
MP = 2
SEM = 128
NCLS = 16
E = 320000
T = 5000

_NBLK = 10
_BN = N // _NBLK
_CH = 2500
_NCH = E // _CH
_UNROLL = 50
_FW = D + 2 * H


def _proj_kernel(x_ref, w_ref, b_ref, natt_ref, map_ref, f_ref):
    m = pl.program_id(0)
    x = x_ref[...]
    w = w_ref[...]
    proj = jnp.dot(x, w, preferred_element_type=jnp.float32) + b_ref[0, :][None, :]
    mask = map_ref[...] == 0
    feat = jnp.where(mask, proj, 0.0)
    natt = natt_ref[pl.ds(2 * m, H), :]
    a_dst = natt[:, 0:D]
    a_src = natt[:, D:2 * D]
    sd = jax.lax.dot_general(feat, a_dst, (((1,), (1,)), ((), ())),
                             preferred_element_type=jnp.float32)
    ss = jax.lax.dot_general(feat, a_src, (((1,), (1,)), ((), ())),
                             preferred_element_type=jnp.float32)
    f_ref[0] = jnp.concatenate([feat, sd, ss], axis=1)


def _scatter_kernel(f_ref, cur_ref, nbr_ref, acc_a_ref, acc_b_ref):
    m = pl.program_id(0)

    @pl.when(pl.program_id(1) == 0)
    def _init():
        acc_a_ref[...] = jnp.zeros_like(acc_a_ref)
        acc_b_ref[...] = jnp.zeros_like(acc_b_ref)

    z = jnp.float32(0.0)

    def edge(i):
        c = cur_ref[0, 0, i]
        nb = nbr_ref[0, 0, i]
        rowc = f_ref[0, c, :]
        rown = f_ref[0, nb, :]
        a0 = rowc[D] + rown[D + 2]
        a1 = rowc[D + 1] + rown[D + 3]
        a0 = jnp.where(a0 >= 0, a0, 0.01 * a0)
        a1 = jnp.where(a1 >= 0, a1, 0.01 * a1)
        ex0 = jnp.exp(a0)
        ex1 = jnp.exp(a1)
        f = rown[0:D]
        va = jnp.concatenate([ex0 * f, ex1 * f])
        acc_a_ref[0, c, :] = acc_a_ref[0, c, :] + va
        vb = jnp.stack([ex0, ex1, z, z, z, z, z, z])
        acc_b_ref[0, c, :] = acc_b_ref[0, c, :] + vb

    def body(j, _):
        for k in range(_UNROLL):
            edge(_UNROLL * j + k)
        return 0

    jax.lax.fori_loop(0, _CH // _UNROLL, body, 0)


def _gather_kernel(acc_a_ref, acc_b_ref, tn_ref, h_ref):
    def body(t, _):
        n = tn_ref[0, 0, t]
        num = acc_a_ref[0, n, :]
        dn = acc_b_ref[0, n, :]
        h0 = num[0:D] / (dn[0] + 1e-16)
        h1 = num[D:2 * D] / (dn[1] + 1e-16)
        h_ref[0, t, :] = jnp.concatenate([h0, h1])
        return 0

    jax.lax.fori_loop(0, T, body, 0)


def _head_kernel(h_ref, wsem_ref, bsem_ref, satt_ref, wcls_ref, bcls_ref,
                 logits_ref, emb_ref):
    h0 = h_ref[0]
    h1 = h_ref[1]
    wsem = wsem_ref[...]
    bsem = bsem_ref[0, :][None, :]
    satt = satt_ref[0, :][None, :]
    a0 = jnp.tanh(jnp.dot(h0, wsem, preferred_element_type=jnp.float32) + bsem)
    a1 = jnp.tanh(jnp.dot(h1, wsem, preferred_element_type=jnp.float32) + bsem)
    l0 = jnp.mean(jnp.sum(satt * a0, axis=1))
    l1 = jnp.mean(jnp.sum(satt * a1, axis=1))
    mx = jnp.maximum(l0, l1)
    e0 = jnp.exp(l0 - mx)
    e1 = jnp.exp(l1 - mx)
    b0 = e0 / (e0 + e1)
    b1 = e1 / (e0 + e1)
    emb = b0 * h0 + b1 * h1
    emb_ref[...] = emb
    logits_ref[...] = (jnp.dot(emb, wcls_ref[...], preferred_element_type=jnp.float32)
                       + bcls_ref[0, :][None, :])


@jax.jit
def kernel(target_nodes, metapath_list, node_type_mapping, node_feature_list,
           W_proj, b_proj, node_attention, W_sem, b_sem, sem_att, W_cls, b_cls):
    x = node_feature_list[0]
    natt4 = node_attention.reshape(MP * H, 2 * D)
    map2 = node_type_mapping.reshape(N, 1)
    b_proj2 = b_proj.reshape(1, D)

    f_table = pl.pallas_call(
        _proj_kernel,
        grid=(MP, _NBLK),
        in_specs=[
            pl.BlockSpec((_BN, DR), lambda m, i: (i, 0)),
            pl.BlockSpec((DR, D), lambda m, i: (0, 0)),
            pl.BlockSpec((1, D), lambda m, i: (0, 0)),
            pl.BlockSpec((MP * H, 2 * D), lambda m, i: (0, 0)),
            pl.BlockSpec((_BN, 1), lambda m, i: (i, 0)),
        ],
        out_specs=pl.BlockSpec((1, _BN, _FW), lambda m, i: (m, i, 0)),
        out_shape=jax.ShapeDtypeStruct((MP, N, _FW), jnp.float32),
        compiler_params=pltpu.CompilerParams(
            dimension_semantics=("parallel", "parallel")),
    )(x, W_proj, b_proj2, natt4, map2)

    cur = metapath_list[:, :, 1].reshape(MP * _NCH, 1, _CH)
    nbr = metapath_list[:, :, 0].reshape(MP * _NCH, 1, _CH)

    acc_a, acc_b = pl.pallas_call(
        _scatter_kernel,
        grid=(MP, _NCH),
        in_specs=[
            pl.BlockSpec((1, N, _FW), lambda m, e: (m, 0, 0)),
            pl.BlockSpec((1, 1, _CH), lambda m, e: (m * _NCH + e, 0, 0),
                         memory_space=pltpu.SMEM),
            pl.BlockSpec((1, 1, _CH), lambda m, e: (m * _NCH + e, 0, 0),
                         memory_space=pltpu.SMEM),
        ],
        out_specs=[
            pl.BlockSpec((1, N, 2 * D), lambda m, e: (m, 0, 0)),
            pl.BlockSpec((1, N, 8), lambda m, e: (m, 0, 0)),
        ],
        out_shape=[
            jax.ShapeDtypeStruct((MP, N, 2 * D), jnp.float32),
            jax.ShapeDtypeStruct((MP, N, 8), jnp.float32),
        ],
        compiler_params=pltpu.CompilerParams(
            dimension_semantics=("parallel", "arbitrary")),
    )(f_table, cur, nbr)

    tn3 = target_nodes.reshape(1, 1, T)
    h_mp = pl.pallas_call(
        _gather_kernel,
        grid=(MP,),
        in_specs=[
            pl.BlockSpec((1, N, 2 * D), lambda m: (m, 0, 0)),
            pl.BlockSpec((1, N, 8), lambda m: (m, 0, 0)),
            pl.BlockSpec((1, 1, T), lambda m: (0, 0, 0),
                         memory_space=pltpu.SMEM),
        ],
        out_specs=pl.BlockSpec((1, T, 2 * D), lambda m: (m, 0, 0)),
        out_shape=jax.ShapeDtypeStruct((MP, T, 2 * D), jnp.float32),
        compiler_params=pltpu.CompilerParams(
            dimension_semantics=("parallel",)),
    )(acc_a, acc_b, tn3)

    logits, emb = pl.pallas_call(
        _head_kernel,
        grid=(1,),
        in_specs=[
            pl.BlockSpec((MP, T, 2 * D), lambda i: (0, 0, 0)),
            pl.BlockSpec((H * D, SEM), lambda i: (0, 0)),
            pl.BlockSpec((1, SEM), lambda i: (0, 0)),
            pl.BlockSpec((1, SEM), lambda i: (0, 0)),
            pl.BlockSpec((H * D, NCLS), lambda i: (0, 0)),
            pl.BlockSpec((1, NCLS), lambda i: (0, 0)),
        ],
        out_specs=[
            pl.BlockSpec((T, NCLS), lambda i: (0, 0)),
            pl.BlockSpec((T, 2 * D), lambda i: (0, 0)),
        ],
        out_shape=[
            jax.ShapeDtypeStruct((T, NCLS), jnp.float32),
            jax.ShapeDtypeStruct((T, 2 * D), jnp.float32),
        ],
    )(h_mp, W_sem, b_sem.reshape(1, SEM), sem_att, W_cls, b_cls.reshape(1, NCLS))

    return (logits, emb)
